# baseline (speedup 1.0000x reference)
import functools

import jax
import jax.numpy as jnp
import numpy as np
from jax import lax
from jax.experimental import pallas as pl
from jax.experimental.pallas import tpu as pltpu
from jax.experimental.pallas import tpu_sc as plsc

_N_LEVELS = 32
_F = 2
_T = 1 << 19
_ENC = 64
_N = 262144

_PRIME1 = -1640531535
_PRIME2 = 805459861
_RES = [float(np.floor(16.0 * 1.3 ** l)) for l in range(_N_LEVELS)]

_NC = 2
_NS = 16
_NW = _NC * _NS
_PPW = _N // _NW
_CHUNK = 256
_NCHUNK = _PPW // _CHUNK
_NGRP = _CHUNK // 16
_NIDX = 8 * _CHUNK
_IDX_ROWS = _NIDX // 128


def _enc_body(xyz_hbm, tab_hbm, res_hbm, enc_hbm,
              xin_v, w_v, idx_v, rows_v, enc_v, res_v, sem):
    cid = lax.axis_index("c")
    sid = lax.axis_index("s")
    wid = sid * _NC + cid
    lane = lax.iota(jnp.int32, 16)
    zero16 = jnp.zeros((16,), jnp.int32)
    one16 = zero16 + 1

    pltpu.sync_copy(res_hbm, res_v)

    def chunk_body(ci, carry):
        base = wid * _PPW + ci * _CHUNK
        pltpu.sync_copy(xyz_hbm.at[pl.ds(base * 3, _CHUNK * 3)], xin_v)

        def xin_body(i, c):
            v = xin_v[pl.ds(i * 16, 16)]
            xin_v[pl.ds(i * 16, 16)] = ((v + 1.0) / 2.0) * 2.0 - 1.0
            return c
        lax.fori_loop(0, (_CHUNK * 3) // 16, xin_body, 0)

        def level_body(lvl, c):
            res = plsc.load_gather(res_v, [zero16 + lvl])
            lvl_base = lvl * _T

            def grp1_body(g, cc):
                p0 = [None] * 3
                for d in range(3):
                    x = plsc.load_gather(xin_v, [lane * 3 + (g * 48 + d)])
                    pos = x * res
                    t = pos.astype(jnp.int32)
                    tf = t.astype(jnp.float32)
                    m = tf > pos
                    p0i = jnp.where(m, t - 1, t)
                    p0f = jnp.where(m, tf - 1.0, tf)
                    w_v[pl.ds(g * 48 + d * 16, 16)] = pos - p0f
                    p0[d] = p0i
                a0 = p0[0]
                a1 = a0 + 1
                b0 = p0[1] * _PRIME1
                b1 = b0 + _PRIME1
                c0 = p0[2] * _PRIME2
                c1 = c0 + _PRIME2
                for o in range(8):
                    i, j, k = (o >> 2) & 1, (o >> 1) & 1, o & 1
                    h = (a1 if i else a0) ^ (b1 if j else b0) ^ (c1 if k else c0)
                    idx = (h & (_T - 1)) + lvl_base
                    idx_v[pl.ds(o * _CHUNK + g * 16, 16)] = idx
                return cc
            lax.fori_loop(0, _NGRP, grp1_body, 0)

            pltpu.async_copy(tab_hbm.at[idx_v], rows_v, sem).wait()

            def grp2_body(g, cc):
                wx = w_v[pl.ds(g * 48, 16)]
                wy = w_v[pl.ds(g * 48 + 16, 16)]
                wz = w_v[pl.ds(g * 48 + 32, 16)]
                ux = 1.0 - wx
                uy = 1.0 - wy
                uz = 1.0 - wz
                acc0 = jnp.zeros((16,), jnp.float32)
                acc1 = jnp.zeros((16,), jnp.float32)
                for o in range(8):
                    i, j, k = (o >> 2) & 1, (o >> 1) & 1, o & 1
                    wt = ((wx if i else ux) * (wy if j else uy)) * (wz if k else uz)
                    ridx = lane + (o * _CHUNK + g * 16)
                    f0 = plsc.load_gather(rows_v, [ridx, zero16])
                    f1 = plsc.load_gather(rows_v, [ridx, one16])
                    acc0 = acc0 + wt * f0
                    acc1 = acc1 + wt * f1
                eidx = (g * 16 + lane) * _ENC + 2 * lvl
                plsc.store_scatter(enc_v, [eidx], acc0)
                plsc.store_scatter(enc_v, [eidx + 1], acc1)
                return cc
            lax.fori_loop(0, _NGRP, grp2_body, 0)
            return c
        lax.fori_loop(0, _N_LEVELS, level_body, 0)

        pltpu.sync_copy(enc_v, enc_hbm.at[pl.ds(base * _ENC, _CHUNK * _ENC)])
        return carry
    lax.fori_loop(0, _NCHUNK, chunk_body, 0)


_encode = functools.partial(
    pl.kernel,
    out_type=jax.ShapeDtypeStruct((_N * _ENC,), jnp.float32),
    mesh=plsc.VectorSubcoreMesh(core_axis_name="c", subcore_axis_name="s"),
    compiler_params=pltpu.CompilerParams(
        needs_layout_passes=False, use_tc_tiling_on_sc=False),
    scratch_types=[
        pltpu.VMEM((_CHUNK * 3,), jnp.float32),
        pltpu.VMEM((_CHUNK * 3,), jnp.float32),
        pltpu.VMEM((_NIDX,), jnp.int32),
        pltpu.VMEM((_NIDX, _F), jnp.float32),
        pltpu.VMEM((_CHUNK * _ENC,), jnp.float32),
        pltpu.VMEM((128,), jnp.float32),
        pltpu.SemaphoreType.DMA,
    ],
)(_enc_body)


_BLK = 2048


def _mlp_body(enc_ref, w0_ref, w1_ref, w2_ref, out_ref):
    h = jnp.maximum(
        jnp.dot(enc_ref[...], w0_ref[...], preferred_element_type=jnp.float32), 0.0)
    h = jnp.maximum(
        jnp.dot(h, w1_ref[...], preferred_element_type=jnp.float32), 0.0)
    y = jnp.dot(h, w2_ref[...], preferred_element_type=jnp.float32)
    s = jax.nn.sigmoid(y)
    col = lax.broadcasted_iota(jnp.int32, s.shape, 1)
    out_ref[...] = jnp.where(col == 3, 0.1 + 0.9 * s, s)


def _mlp(enc, W0, W1, W2p):
    return pl.pallas_call(
        _mlp_body,
        grid=(_N // _BLK,),
        in_specs=[
            pl.BlockSpec((_BLK, _ENC), lambda i: (i, 0)),
            pl.BlockSpec((_ENC, 64), lambda i: (0, 0)),
            pl.BlockSpec((64, 64), lambda i: (0, 0)),
            pl.BlockSpec((64, 8), lambda i: (0, 0)),
        ],
        out_specs=pl.BlockSpec((_BLK, 8), lambda i: (i, 0)),
        out_shape=jax.ShapeDtypeStruct((_N, 8), jnp.float32),
    )(enc, W0, W1, W2p)


def kernel(xyz, table, W0, W1, W2):
    xyz_flat = xyz.reshape(-1)
    tab2 = table.reshape(_N_LEVELS * _T, _F)
    res_arr = jnp.zeros((128,), jnp.float32).at[:_N_LEVELS].set(
        jnp.asarray(_RES, jnp.float32))
    enc = _encode(xyz_flat, tab2, res_arr).reshape(_N, _ENC)
    W2p = jnp.pad(W2, ((0, 0), (0, 3)))
    out = _mlp(enc, W0, W1, W2p)
    return (out[:, 0:3], out[:, 3:4], out[:, 4:5])


# speedup vs baseline: 4.0861x; 4.0861x over previous
import functools

import jax
import jax.numpy as jnp
import numpy as np
from jax import lax
from jax.experimental import pallas as pl
from jax.experimental.pallas import tpu as pltpu
from jax.experimental.pallas import tpu_sc as plsc

_N_LEVELS = 32
_F = 2
_T = 1 << 19
_ENC = 64
_N = 262144

_PRIME1 = -1640531535
_PRIME2 = 805459861
_RES = [float(np.floor(16.0 * 1.3 ** l)) for l in range(_N_LEVELS)]

_NC = 2
_NS = 16
_NW = _NC * _NS
_PPW = _N // _NW
_CHUNK = 256
_NCHUNK = _PPW // _CHUNK
_NGRP = _CHUNK // 16
_NIDX = 8 * _CHUNK
_IDX_ROWS = _NIDX // 128


def _enc_body(xyz_hbm, tab0_hbm, tab1_hbm, res_hbm, enc_hbm,
              xin_v, w_v, idx_v, rows0_v, rows1_v, enc_v, res_v, sem):
    cid = lax.axis_index("c")
    sid = lax.axis_index("s")
    wid = sid * _NC + cid
    lane = lax.iota(jnp.int32, 16)
    zero16 = jnp.zeros((16,), jnp.int32)
    one16 = zero16 + 1

    pltpu.sync_copy(res_hbm, res_v)

    def chunk_body(ci, carry):
        base = wid * _PPW + ci * _CHUNK
        pltpu.sync_copy(xyz_hbm.at[pl.ds(base * 3, _CHUNK * 3)], xin_v)

        def xin_body(i, c):
            v = xin_v[pl.ds(i * 16, 16)]
            xin_v[pl.ds(i * 16, 16)] = ((v + 1.0) / 2.0) * 2.0 - 1.0
            return c
        lax.fori_loop(0, (_CHUNK * 3) // 16, xin_body, 0)

        def level_body(lvl, c):
            res = plsc.load_gather(res_v, [zero16 + lvl])
            lvl_base = lvl * _T

            def grp1_body(g, cc):
                p0 = [None] * 3
                for d in range(3):
                    x = plsc.load_gather(xin_v, [lane * 3 + (g * 48 + d)])
                    pos = x * res
                    t = pos.astype(jnp.int32)
                    tf = t.astype(jnp.float32)
                    m = tf > pos
                    p0i = jnp.where(m, t - 1, t)
                    p0f = jnp.where(m, tf - 1.0, tf)
                    w_v[pl.ds(g * 48 + d * 16, 16)] = pos - p0f
                    p0[d] = p0i
                a0 = p0[0]
                a1 = a0 + 1
                b0 = p0[1] * _PRIME1
                b1 = b0 + _PRIME1
                c0 = p0[2] * _PRIME2
                c1 = c0 + _PRIME2
                for o in range(8):
                    i, j, k = (o >> 2) & 1, (o >> 1) & 1, o & 1
                    h = (a1 if i else a0) ^ (b1 if j else b0) ^ (c1 if k else c0)
                    idx = (h & (_T - 1)) + lvl_base
                    idx_v[pl.ds(o * _CHUNK + g * 16, 16)] = idx
                return cc
            lax.fori_loop(0, _NGRP, grp1_body, 0)

            cp0 = pltpu.async_copy(tab0_hbm.at[idx_v], rows0_v, sem)
            cp1 = pltpu.async_copy(tab1_hbm.at[idx_v], rows1_v, sem)
            cp0.wait()
            cp1.wait()

            def grp2_body(g, cc):
                wx = w_v[pl.ds(g * 48, 16)]
                wy = w_v[pl.ds(g * 48 + 16, 16)]
                wz = w_v[pl.ds(g * 48 + 32, 16)]
                ux = 1.0 - wx
                uy = 1.0 - wy
                uz = 1.0 - wz
                acc0 = jnp.zeros((16,), jnp.float32)
                acc1 = jnp.zeros((16,), jnp.float32)
                for o in range(8):
                    i, j, k = (o >> 2) & 1, (o >> 1) & 1, o & 1
                    wt = ((wx if i else ux) * (wy if j else uy)) * (wz if k else uz)
                    f0 = rows0_v[pl.ds(o * _CHUNK + g * 16, 16)]
                    f1 = rows1_v[pl.ds(o * _CHUNK + g * 16, 16)]
                    acc0 = acc0 + wt * f0
                    acc1 = acc1 + wt * f1
                eidx = (g * 16 + lane) * _ENC + 2 * lvl
                plsc.store_scatter(enc_v, [eidx], acc0)
                plsc.store_scatter(enc_v, [eidx + 1], acc1)
                return cc
            lax.fori_loop(0, _NGRP, grp2_body, 0)
            return c
        lax.fori_loop(0, _N_LEVELS, level_body, 0)

        pltpu.sync_copy(enc_v, enc_hbm.at[pl.ds(base * _ENC, _CHUNK * _ENC)])
        return carry
    lax.fori_loop(0, _NCHUNK, chunk_body, 0)


_encode = functools.partial(
    pl.kernel,
    out_type=jax.ShapeDtypeStruct((_N * _ENC,), jnp.float32),
    mesh=plsc.VectorSubcoreMesh(core_axis_name="c", subcore_axis_name="s"),
    compiler_params=pltpu.CompilerParams(
        needs_layout_passes=False, use_tc_tiling_on_sc=False),
    scratch_types=[
        pltpu.VMEM((_CHUNK * 3,), jnp.float32),
        pltpu.VMEM((_CHUNK * 3,), jnp.float32),
        pltpu.VMEM((_NIDX,), jnp.int32),
        pltpu.VMEM((_NIDX,), jnp.float32),
        pltpu.VMEM((_NIDX,), jnp.float32),
        pltpu.VMEM((_CHUNK * _ENC,), jnp.float32),
        pltpu.VMEM((128,), jnp.float32),
        pltpu.SemaphoreType.DMA,
    ],
)(_enc_body)


_BLK = 2048


def _mlp_body(enc_ref, w0_ref, w1_ref, w2_ref, out_ref):
    h = jnp.maximum(
        jnp.dot(enc_ref[...], w0_ref[...], preferred_element_type=jnp.float32), 0.0)
    h = jnp.maximum(
        jnp.dot(h, w1_ref[...], preferred_element_type=jnp.float32), 0.0)
    y = jnp.dot(h, w2_ref[...], preferred_element_type=jnp.float32)
    s = jax.nn.sigmoid(y)
    col = lax.broadcasted_iota(jnp.int32, s.shape, 1)
    out_ref[...] = jnp.where(col == 3, 0.1 + 0.9 * s, s)


def _mlp(enc, W0, W1, W2p):
    return pl.pallas_call(
        _mlp_body,
        grid=(_N // _BLK,),
        in_specs=[
            pl.BlockSpec((_BLK, _ENC), lambda i: (i, 0)),
            pl.BlockSpec((_ENC, 64), lambda i: (0, 0)),
            pl.BlockSpec((64, 64), lambda i: (0, 0)),
            pl.BlockSpec((64, 8), lambda i: (0, 0)),
        ],
        out_specs=pl.BlockSpec((_BLK, 8), lambda i: (i, 0)),
        out_shape=jax.ShapeDtypeStruct((_N, 8), jnp.float32),
    )(enc, W0, W1, W2p)


def kernel(xyz, table, W0, W1, W2):
    xyz_flat = xyz.reshape(-1)
    tab0 = table[:, :, 0].reshape(-1)
    tab1 = table[:, :, 1].reshape(-1)
    res_arr = jnp.zeros((128,), jnp.float32).at[:_N_LEVELS].set(
        jnp.asarray(_RES, jnp.float32))
    enc = _encode(xyz_flat, tab0, tab1, res_arr).reshape(_N, _ENC)
    W2p = jnp.pad(W2, ((0, 0), (0, 3)))
    out = _mlp(enc, W0, W1, W2p)
    return (out[:, 0:3], out[:, 3:4], out[:, 4:5])


# speedup vs baseline: 5.7748x; 1.4133x over previous
import functools

import jax
import jax.numpy as jnp
import numpy as np
from jax import lax
from jax.experimental import pallas as pl
from jax.experimental.pallas import tpu as pltpu
from jax.experimental.pallas import tpu_sc as plsc

_N_LEVELS = 32
_F = 2
_T = 1 << 19
_ENC = 64
_N = 262144

_PRIME1 = -1640531535
_PRIME2 = 805459861
_RES = [float(np.floor(16.0 * 1.3 ** l)) for l in range(_N_LEVELS)]

_NC = 2
_NS = 16
_NW = _NC * _NS
_PPW = _N // _NW
_CHUNK = 256
_NCHUNK = _PPW // _CHUNK
_NGRP = _CHUNK // 16
_NIDX = 8 * _CHUNK
_IDX_ROWS = _NIDX // 128


def _enc_body(xyz_hbm, tab_hbm, res_hbm, enc_hbm,
              xin_v, w_v, idx_v, rows_v, enc_v, res_v, sem):
    cid = lax.axis_index("c")
    sid = lax.axis_index("s")
    wid = sid * _NC + cid
    lane = lax.iota(jnp.int32, 16)
    zero16 = jnp.zeros((16,), jnp.int32)
    one16 = zero16 + 1

    pltpu.sync_copy(res_hbm, res_v)

    def chunk_body(ci, carry):
        base = wid * _PPW + ci * _CHUNK
        pltpu.sync_copy(xyz_hbm.at[pl.ds(base * 3, _CHUNK * 3)], xin_v)

        def xin_body(i, c):
            v = xin_v[pl.ds(i * 16, 16)]
            xin_v[pl.ds(i * 16, 16)] = ((v + 1.0) / 2.0) * 2.0 - 1.0
            return c
        lax.fori_loop(0, (_CHUNK * 3) // 16, xin_body, 0)

        def level_body(lvl, c):
            res = plsc.load_gather(res_v, [zero16 + lvl])
            lvl_base = lvl * _T

            def grp1_body(g, cc):
                p0 = [None] * 3
                for d in range(3):
                    x = plsc.load_gather(xin_v, [lane * 3 + (g * 48 + d)])
                    pos = x * res
                    t = pos.astype(jnp.int32)
                    tf = t.astype(jnp.float32)
                    m = tf > pos
                    p0i = jnp.where(m, t - 1, t)
                    p0f = jnp.where(m, tf - 1.0, tf)
                    w_v[pl.ds(g * 48 + d * 16, 16)] = pos - p0f
                    p0[d] = p0i
                a0 = p0[0]
                a1 = a0 + 1
                b0 = p0[1] * _PRIME1
                b1 = b0 + _PRIME1
                c0 = p0[2] * _PRIME2
                c1 = c0 + _PRIME2
                for o in range(8):
                    i, j, k = (o >> 2) & 1, (o >> 1) & 1, o & 1
                    h = (a1 if i else a0) ^ (b1 if j else b0) ^ (c1 if k else c0)
                    idx = (h & (_T - 1)) + lvl_base
                    idx_v[pl.ds(o * _CHUNK + g * 16, 16)] = idx
                return cc
            lax.fori_loop(0, _NGRP, grp1_body, 0)

            pltpu.async_copy(tab_hbm.at[idx_v], rows_v, sem).wait()

            def grp2_body(g, cc):
                wx = w_v[pl.ds(g * 48, 16)]
                wy = w_v[pl.ds(g * 48 + 16, 16)]
                wz = w_v[pl.ds(g * 48 + 32, 16)]
                ux = 1.0 - wx
                uy = 1.0 - wy
                uz = 1.0 - wz
                acc0 = jnp.zeros((16,), jnp.float32)
                acc1 = jnp.zeros((16,), jnp.float32)
                for o in range(8):
                    i, j, k = (o >> 2) & 1, (o >> 1) & 1, o & 1
                    wt = ((wx if i else ux) * (wy if j else uy)) * (wz if k else uz)
                    fw = rows_v[pl.ds(o * _CHUNK + g * 16, 16)]
                    f0 = plsc.bitcast(fw << 16, jnp.float32)
                    f1 = plsc.bitcast(fw & -65536, jnp.float32)
                    acc0 = acc0 + wt * f0
                    acc1 = acc1 + wt * f1
                eidx = (g * 16 + lane) * _ENC + 2 * lvl
                plsc.store_scatter(enc_v, [eidx], acc0)
                plsc.store_scatter(enc_v, [eidx + 1], acc1)
                return cc
            lax.fori_loop(0, _NGRP, grp2_body, 0)
            return c
        lax.fori_loop(0, _N_LEVELS, level_body, 0)

        pltpu.sync_copy(enc_v, enc_hbm.at[pl.ds(base * _ENC, _CHUNK * _ENC)])
        return carry
    lax.fori_loop(0, _NCHUNK, chunk_body, 0)


_encode = functools.partial(
    pl.kernel,
    out_type=jax.ShapeDtypeStruct((_N * _ENC,), jnp.float32),
    mesh=plsc.VectorSubcoreMesh(core_axis_name="c", subcore_axis_name="s"),
    compiler_params=pltpu.CompilerParams(
        needs_layout_passes=False, use_tc_tiling_on_sc=False),
    scratch_types=[
        pltpu.VMEM((_CHUNK * 3,), jnp.float32),
        pltpu.VMEM((_CHUNK * 3,), jnp.float32),
        pltpu.VMEM((_NIDX,), jnp.int32),
        pltpu.VMEM((_NIDX,), jnp.int32),
        pltpu.VMEM((_CHUNK * _ENC,), jnp.float32),
        pltpu.VMEM((128,), jnp.float32),
        pltpu.SemaphoreType.DMA,
    ],
)(_enc_body)


_BLK = 2048


def _mlp_body(enc_ref, w0_ref, w1_ref, w2_ref, out_ref):
    h = jnp.maximum(
        jnp.dot(enc_ref[...], w0_ref[...], preferred_element_type=jnp.float32), 0.0)
    h = jnp.maximum(
        jnp.dot(h, w1_ref[...], preferred_element_type=jnp.float32), 0.0)
    y = jnp.dot(h, w2_ref[...], preferred_element_type=jnp.float32)
    s = jax.nn.sigmoid(y)
    col = lax.broadcasted_iota(jnp.int32, s.shape, 1)
    out_ref[...] = jnp.where(col == 3, 0.1 + 0.9 * s, s)


def _mlp(enc, W0, W1, W2p):
    return pl.pallas_call(
        _mlp_body,
        grid=(_N // _BLK,),
        in_specs=[
            pl.BlockSpec((_BLK, _ENC), lambda i: (i, 0)),
            pl.BlockSpec((_ENC, 64), lambda i: (0, 0)),
            pl.BlockSpec((64, 64), lambda i: (0, 0)),
            pl.BlockSpec((64, 8), lambda i: (0, 0)),
        ],
        out_specs=pl.BlockSpec((_BLK, 8), lambda i: (i, 0)),
        out_shape=jax.ShapeDtypeStruct((_N, 8), jnp.float32),
    )(enc, W0, W1, W2p)


def kernel(xyz, table, W0, W1, W2):
    xyz_flat = xyz.reshape(-1)
    tabp = lax.bitcast_convert_type(
        table.astype(jnp.bfloat16), jnp.int32).reshape(-1)
    res_arr = jnp.zeros((128,), jnp.float32).at[:_N_LEVELS].set(
        jnp.asarray(_RES, jnp.float32))
    enc = _encode(xyz_flat, tabp, res_arr).reshape(_N, _ENC)
    W2p = jnp.pad(W2, ((0, 0), (0, 3)))
    out = _mlp(enc, W0, W1, W2p)
    return (out[:, 0:3], out[:, 3:4], out[:, 4:5])


# speedup vs baseline: 8.9342x; 1.5471x over previous
import functools

import jax
import jax.numpy as jnp
import numpy as np
from jax import lax
from jax.experimental import pallas as pl
from jax.experimental.pallas import tpu as pltpu
from jax.experimental.pallas import tpu_sc as plsc

_N_LEVELS = 32
_F = 2
_T = 1 << 19
_ENC = 64
_N = 262144

_PRIME1 = -1640531535
_PRIME2 = 805459861
_RES = [float(np.floor(16.0 * 1.3 ** l)) for l in range(_N_LEVELS)]

_NC = 2
_NS = 16
_NW = _NC * _NS
_PPW = _N // _NW
_CHUNK = 256
_NCHUNK = _PPW // _CHUNK
_NGRP = _CHUNK // 16
_NIDX = 8 * _CHUNK
_IDX_ROWS = _NIDX // 128


def _enc_body(xyz_hbm, tab_hbm, res_hbm, enc_hbm,
              xin_v, wA_v, wB_v, idxA_v, idxB_v, rowsA_v, rowsB_v,
              enc_v, res_v, semA, semB):
    cid = lax.axis_index("c")
    sid = lax.axis_index("s")
    wid = sid * _NC + cid
    lane = lax.iota(jnp.int32, 16)
    zero16 = jnp.zeros((16,), jnp.int32)

    pltpu.sync_copy(res_hbm, res_v)

    def pass1(lvl, w_v, idx_v):
        res = plsc.load_gather(res_v, [zero16 + lvl])
        lvl_base = lvl * _T

        def grp1_body(g, cc):
            p0 = [None] * 3
            for d in range(3):
                x = plsc.load_gather(xin_v, [lane * 3 + (g * 48 + d)])
                pos = x * res
                t = pos.astype(jnp.int32)
                tf = t.astype(jnp.float32)
                m = tf > pos
                p0i = jnp.where(m, t - 1, t)
                p0f = jnp.where(m, tf - 1.0, tf)
                w_v[pl.ds(g * 48 + d * 16, 16)] = pos - p0f
                p0[d] = p0i
            a0 = p0[0]
            a1 = a0 + 1
            b0 = p0[1] * _PRIME1
            b1 = b0 + _PRIME1
            c0 = p0[2] * _PRIME2
            c1 = c0 + _PRIME2
            for o in range(8):
                i, j, k = (o >> 2) & 1, (o >> 1) & 1, o & 1
                h = (a1 if i else a0) ^ (b1 if j else b0) ^ (c1 if k else c0)
                idx = (h & (_T - 1)) + lvl_base
                idx_v[pl.ds(o * _CHUNK + g * 16, 16)] = idx
            return cc
        lax.fori_loop(0, _NGRP, grp1_body, 0)

    def fire(idx_v, rows_v, sem):
        pltpu.make_async_copy(tab_hbm.at[idx_v], rows_v, sem).start()

    def drain(idx_v, rows_v, sem):
        pltpu.make_async_copy(tab_hbm.at[idx_v], rows_v, sem).wait()

    def pass2(lvl, w_v, rows_v):
        def grp2_body(g, cc):
            wx = w_v[pl.ds(g * 48, 16)]
            wy = w_v[pl.ds(g * 48 + 16, 16)]
            wz = w_v[pl.ds(g * 48 + 32, 16)]
            ux = 1.0 - wx
            uy = 1.0 - wy
            uz = 1.0 - wz
            acc0 = jnp.zeros((16,), jnp.float32)
            acc1 = jnp.zeros((16,), jnp.float32)
            for o in range(8):
                i, j, k = (o >> 2) & 1, (o >> 1) & 1, o & 1
                wt = ((wx if i else ux) * (wy if j else uy)) * (wz if k else uz)
                fw = rows_v[pl.ds(o * _CHUNK + g * 16, 16)]
                f0 = plsc.bitcast(fw << 16, jnp.float32)
                f1 = plsc.bitcast(fw & -65536, jnp.float32)
                acc0 = acc0 + wt * f0
                acc1 = acc1 + wt * f1
            eidx = (g * 16 + lane) * _ENC + 2 * lvl
            plsc.store_scatter(enc_v, [eidx], acc0)
            plsc.store_scatter(enc_v, [eidx + 1], acc1)
            return cc
        lax.fori_loop(0, _NGRP, grp2_body, 0)

    def chunk_body(ci, carry):
        base = wid * _PPW + ci * _CHUNK
        pltpu.sync_copy(xyz_hbm.at[pl.ds(base * 3, _CHUNK * 3)], xin_v)

        def xin_body(i, c):
            v = xin_v[pl.ds(i * 16, 16)]
            xin_v[pl.ds(i * 16, 16)] = ((v + 1.0) / 2.0) * 2.0 - 1.0
            return c
        lax.fori_loop(0, (_CHUNK * 3) // 16, xin_body, 0)

        pass1(0, wA_v, idxA_v)
        fire(idxA_v, rowsA_v, semA)

        def pair_body(ii, c):
            lvl = 2 * ii
            pass1(lvl + 1, wB_v, idxB_v)
            fire(idxB_v, rowsB_v, semB)
            drain(idxA_v, rowsA_v, semA)
            pass2(lvl, wA_v, rowsA_v)
            pass1(lvl + 2, wA_v, idxA_v)
            fire(idxA_v, rowsA_v, semA)
            drain(idxB_v, rowsB_v, semB)
            pass2(lvl + 1, wB_v, rowsB_v)
            return c
        lax.fori_loop(0, _N_LEVELS // 2 - 1, pair_body, 0)

        pass1(_N_LEVELS - 1, wB_v, idxB_v)
        fire(idxB_v, rowsB_v, semB)
        drain(idxA_v, rowsA_v, semA)
        pass2(_N_LEVELS - 2, wA_v, rowsA_v)
        drain(idxB_v, rowsB_v, semB)
        pass2(_N_LEVELS - 1, wB_v, rowsB_v)

        pltpu.sync_copy(enc_v, enc_hbm.at[pl.ds(base * _ENC, _CHUNK * _ENC)])
        return carry
    lax.fori_loop(0, _NCHUNK, chunk_body, 0)


_encode = functools.partial(
    pl.kernel,
    out_type=jax.ShapeDtypeStruct((_N * _ENC,), jnp.float32),
    mesh=plsc.VectorSubcoreMesh(core_axis_name="c", subcore_axis_name="s"),
    compiler_params=pltpu.CompilerParams(
        needs_layout_passes=False, use_tc_tiling_on_sc=False),
    scratch_types=[
        pltpu.VMEM((_CHUNK * 3,), jnp.float32),
        pltpu.VMEM((_CHUNK * 3,), jnp.float32),
        pltpu.VMEM((_CHUNK * 3,), jnp.float32),
        pltpu.VMEM((_NIDX,), jnp.int32),
        pltpu.VMEM((_NIDX,), jnp.int32),
        pltpu.VMEM((_NIDX,), jnp.int32),
        pltpu.VMEM((_NIDX,), jnp.int32),
        pltpu.VMEM((_CHUNK * _ENC,), jnp.float32),
        pltpu.VMEM((128,), jnp.float32),
        pltpu.SemaphoreType.DMA,
        pltpu.SemaphoreType.DMA,
    ],
)(_enc_body)


_BLK = 2048


def _mlp_body(enc_ref, w0_ref, w1_ref, w2_ref, out_ref):
    h = jnp.maximum(
        jnp.dot(enc_ref[...], w0_ref[...], preferred_element_type=jnp.float32), 0.0)
    h = jnp.maximum(
        jnp.dot(h, w1_ref[...], preferred_element_type=jnp.float32), 0.0)
    y = jnp.dot(h, w2_ref[...], preferred_element_type=jnp.float32)
    s = jax.nn.sigmoid(y)
    col = lax.broadcasted_iota(jnp.int32, s.shape, 1)
    out_ref[...] = jnp.where(col == 3, 0.1 + 0.9 * s, s)


def _mlp(enc, W0, W1, W2p):
    return pl.pallas_call(
        _mlp_body,
        grid=(_N // _BLK,),
        in_specs=[
            pl.BlockSpec((_BLK, _ENC), lambda i: (i, 0)),
            pl.BlockSpec((_ENC, 64), lambda i: (0, 0)),
            pl.BlockSpec((64, 64), lambda i: (0, 0)),
            pl.BlockSpec((64, 8), lambda i: (0, 0)),
        ],
        out_specs=pl.BlockSpec((_BLK, 8), lambda i: (i, 0)),
        out_shape=jax.ShapeDtypeStruct((_N, 8), jnp.float32),
    )(enc, W0, W1, W2p)


def kernel(xyz, table, W0, W1, W2):
    xyz_flat = xyz.reshape(-1)
    tabp = lax.bitcast_convert_type(
        table.astype(jnp.bfloat16), jnp.int32).reshape(-1)
    res_arr = jnp.zeros((128,), jnp.float32).at[:_N_LEVELS].set(
        jnp.asarray(_RES, jnp.float32))
    enc = _encode(xyz_flat, tabp, res_arr).reshape(_N, _ENC)
    W2p = jnp.pad(W2, ((0, 0), (0, 3)))
    out = _mlp(enc, W0, W1, W2p)
    return (out[:, 0:3], out[:, 3:4], out[:, 4:5])


# speedup vs baseline: 9.0540x; 1.0134x over previous
import functools

import jax
import jax.numpy as jnp
import numpy as np
from jax import lax
from jax.experimental import pallas as pl
from jax.experimental.pallas import tpu as pltpu
from jax.experimental.pallas import tpu_sc as plsc

_N_LEVELS = 32
_F = 2
_T = 1 << 19
_ENC = 64
_N = 262144

_PRIME1 = -1640531535
_PRIME2 = 805459861
_RES = [float(np.floor(16.0 * 1.3 ** l)) for l in range(_N_LEVELS)]

_NC = 2
_NS = 16
_NW = _NC * _NS
_PPW = _N // _NW
_CHUNK = 256
_NCHUNK = _PPW // _CHUNK
_NGRP = _CHUNK // 16
_NIDX = 8 * _CHUNK
_IDX_ROWS = _NIDX // 128


def _enc_body(xyz_hbm, tab_hbm, res_hbm, enc_hbm,
              xin_v, wA_v, wB_v, idxA_v, idxB_v, rowsA_v, rowsB_v,
              enc_v, res_v, semA, semB):
    cid = lax.axis_index("c")
    sid = lax.axis_index("s")
    wid = sid * _NC + cid
    lane = lax.iota(jnp.int32, 16)
    zero16 = jnp.zeros((16,), jnp.int32)

    pltpu.sync_copy(res_hbm, res_v)

    def pass1(lvl, w_v, idx_v):
        res = plsc.load_gather(res_v, [zero16 + lvl])
        lvl_base = lvl * _T

        def grp1_body(g, cc):
            p0 = [None] * 3
            for d in range(3):
                x = plsc.load_gather(xin_v, [lane * 3 + (g * 48 + d)])
                pos = x * res
                t = pos.astype(jnp.int32)
                tf = t.astype(jnp.float32)
                m = tf > pos
                p0i = jnp.where(m, t - 1, t)
                p0f = jnp.where(m, tf - 1.0, tf)
                w_v[pl.ds(g * 48 + d * 16, 16)] = pos - p0f
                p0[d] = p0i
            a0 = p0[0]
            a1 = a0 + 1
            b0 = p0[1] * _PRIME1
            b1 = b0 + _PRIME1
            c0 = p0[2] * _PRIME2
            c1 = c0 + _PRIME2
            for o in range(8):
                i, j, k = (o >> 2) & 1, (o >> 1) & 1, o & 1
                h = (a1 if i else a0) ^ (b1 if j else b0) ^ (c1 if k else c0)
                idx = (h & (_T - 1)) + lvl_base
                idx_v[pl.ds(o * _CHUNK + g * 16, 16)] = idx
            return cc
        lax.fori_loop(0, _NGRP, grp1_body, 0)

    def fire(idx_v, rows_v, sem):
        pltpu.make_async_copy(tab_hbm.at[idx_v], rows_v, sem).start()

    def drain(idx_v, rows_v, sem):
        pltpu.make_async_copy(tab_hbm.at[idx_v], rows_v, sem).wait()

    def pass2(lvl, w_v, rows_v):
        def grp2_body(g, cc):
            wx = w_v[pl.ds(g * 48, 16)]
            wy = w_v[pl.ds(g * 48 + 16, 16)]
            wz = w_v[pl.ds(g * 48 + 32, 16)]
            ux = 1.0 - wx
            uy = 1.0 - wy
            uz = 1.0 - wz
            acc0 = jnp.zeros((16,), jnp.float32)
            acc1 = jnp.zeros((16,), jnp.float32)
            for o in range(8):
                i, j, k = (o >> 2) & 1, (o >> 1) & 1, o & 1
                wt = ((wx if i else ux) * (wy if j else uy)) * (wz if k else uz)
                fw = rows_v[pl.ds(o * _CHUNK + g * 16, 16)]
                f0 = plsc.bitcast(fw << 16, jnp.float32)
                f1 = plsc.bitcast(fw & -65536, jnp.float32)
                acc0 = acc0 + wt * f0
                acc1 = acc1 + wt * f1
            eidx = (g * 16 + lane) * _ENC + 2 * lvl
            plsc.store_scatter(enc_v, [eidx], acc0)
            plsc.store_scatter(enc_v, [eidx + 1], acc1)
            return cc
        lax.fori_loop(0, _NGRP, grp2_body, 0)

    def chunk_body(ci, carry):
        base = wid * _PPW + ci * _CHUNK
        pltpu.sync_copy(xyz_hbm.at[pl.ds(base * 3, _CHUNK * 3)], xin_v)

        def xin_body(i, c):
            v = xin_v[pl.ds(i * 16, 16)]
            xin_v[pl.ds(i * 16, 16)] = ((v + 1.0) / 2.0) * 2.0 - 1.0
            return c
        lax.fori_loop(0, (_CHUNK * 3) // 16, xin_body, 0)

        pass1(0, wA_v, idxA_v)
        fire(idxA_v, rowsA_v, semA)

        def pair_body(ii, c):
            lvl = 2 * ii
            pass1(lvl + 1, wB_v, idxB_v)
            fire(idxB_v, rowsB_v, semB)
            drain(idxA_v, rowsA_v, semA)
            pass2(lvl, wA_v, rowsA_v)
            pass1(lvl + 2, wA_v, idxA_v)
            fire(idxA_v, rowsA_v, semA)
            drain(idxB_v, rowsB_v, semB)
            pass2(lvl + 1, wB_v, rowsB_v)
            return c
        lax.fori_loop(0, _N_LEVELS // 2 - 1, pair_body, 0)

        pass1(_N_LEVELS - 1, wB_v, idxB_v)
        fire(idxB_v, rowsB_v, semB)
        drain(idxA_v, rowsA_v, semA)
        pass2(_N_LEVELS - 2, wA_v, rowsA_v)
        drain(idxB_v, rowsB_v, semB)
        pass2(_N_LEVELS - 1, wB_v, rowsB_v)

        pltpu.sync_copy(enc_v, enc_hbm.at[pl.ds(base * _ENC, _CHUNK * _ENC)])
        return carry
    lax.fori_loop(0, _NCHUNK, chunk_body, 0)


_encode = functools.partial(
    pl.kernel,
    out_type=jax.ShapeDtypeStruct((_N * _ENC,), jnp.float32),
    mesh=plsc.VectorSubcoreMesh(core_axis_name="c", subcore_axis_name="s"),
    compiler_params=pltpu.CompilerParams(
        needs_layout_passes=False, use_tc_tiling_on_sc=False),
    scratch_types=[
        pltpu.VMEM((_CHUNK * 3,), jnp.float32),
        pltpu.VMEM((_CHUNK * 3,), jnp.float32),
        pltpu.VMEM((_CHUNK * 3,), jnp.float32),
        pltpu.VMEM((_NIDX,), jnp.int32),
        pltpu.VMEM((_NIDX,), jnp.int32),
        pltpu.VMEM((_NIDX,), jnp.int32),
        pltpu.VMEM((_NIDX,), jnp.int32),
        pltpu.VMEM((_CHUNK * _ENC,), jnp.float32),
        pltpu.VMEM((128,), jnp.float32),
        pltpu.SemaphoreType.DMA,
        pltpu.SemaphoreType.DMA,
    ],
)(_enc_body)


_BLK = 2048


def _mlp_body(enc_ref, w0_ref, w1_ref, w2_ref, out_ref):
    h = jnp.maximum(
        jnp.dot(enc_ref[...], w0_ref[...], preferred_element_type=jnp.float32), 0.0)
    h = jnp.maximum(
        jnp.dot(h, w1_ref[...], preferred_element_type=jnp.float32), 0.0)
    y = jnp.dot(h, w2_ref[...], preferred_element_type=jnp.float32)
    s = jax.nn.sigmoid(y)
    col = lax.broadcasted_iota(jnp.int32, s.shape, 1)
    out_ref[...] = jnp.where(col == 3, 0.1 + 0.9 * s, s)


def _mlp(enc, W0, W1, W2p):
    return pl.pallas_call(
        _mlp_body,
        grid=(_N // _BLK,),
        in_specs=[
            pl.BlockSpec((_BLK, _ENC), lambda i: (i, 0)),
            pl.BlockSpec((_ENC, 64), lambda i: (0, 0)),
            pl.BlockSpec((64, 64), lambda i: (0, 0)),
            pl.BlockSpec((64, 8), lambda i: (0, 0)),
        ],
        out_specs=pl.BlockSpec((_BLK, 8), lambda i: (i, 0)),
        out_shape=jax.ShapeDtypeStruct((_N, 8), jnp.float32),
    )(enc, W0, W1, W2p)


def kernel(xyz, table, W0, W1, W2):
    xyz_flat = xyz.reshape(-1)
    tt = table.transpose(0, 2, 1)
    a = lax.bitcast_convert_type(
        tt[:, 0, :].astype(jnp.bfloat16), jnp.uint16).astype(jnp.uint32)
    b = lax.bitcast_convert_type(
        tt[:, 1, :].astype(jnp.bfloat16), jnp.uint16).astype(jnp.uint32)
    tabp = lax.bitcast_convert_type(a | (b << 16), jnp.int32).reshape(-1)
    res_arr = jnp.zeros((128,), jnp.float32).at[:_N_LEVELS].set(
        jnp.asarray(_RES, jnp.float32))
    enc = _encode(xyz_flat, tabp, res_arr).reshape(_N, _ENC)
    W2p = jnp.pad(W2, ((0, 0), (0, 3)))
    out = _mlp(enc, W0, W1, W2p)
    return (out[:, 0:3], out[:, 3:4], out[:, 4:5])


# speedup vs baseline: 9.0584x; 1.0005x over previous
import functools

import jax
import jax.numpy as jnp
import numpy as np
from jax import lax
from jax.experimental import pallas as pl
from jax.experimental.pallas import tpu as pltpu
from jax.experimental.pallas import tpu_sc as plsc

_N_LEVELS = 32
_F = 2
_T = 1 << 19
_ENC = 64
_N = 262144

_PRIME1 = -1640531535
_PRIME2 = 805459861
_RES = [float(np.floor(16.0 * 1.3 ** l)) for l in range(_N_LEVELS)]

_NC = 2
_NS = 16
_NW = _NC * _NS
_PPW = _N // _NW
_CHUNK = 256
_NCHUNK = _PPW // _CHUNK
_NGRP = _CHUNK // 16
_NIDX = 8 * _CHUNK
_IDX_ROWS = _NIDX // 128


def _enc_body(xyz_hbm, tab_hbm, res_hbm, enc_hbm,
              xin_v, wA_v, wB_v, wC_v, idxA_v, idxB_v, idxC_v,
              rowsA_v, rowsB_v, rowsC_v, enc_v, res_v, semA, semB, semC):
    cid = lax.axis_index("c")
    sid = lax.axis_index("s")
    wid = sid * _NC + cid
    lane = lax.iota(jnp.int32, 16)
    zero16 = jnp.zeros((16,), jnp.int32)

    pltpu.sync_copy(res_hbm, res_v)

    bufs = ((wA_v, idxA_v, rowsA_v, semA),
            (wB_v, idxB_v, rowsB_v, semB),
            (wC_v, idxC_v, rowsC_v, semC))

    def fire(b):
        pltpu.make_async_copy(
            tab_hbm.at[bufs[b][1]], bufs[b][2], bufs[b][3]).start()

    def drain(b):
        pltpu.make_async_copy(
            tab_hbm.at[bufs[b][1]], bufs[b][2], bufs[b][3]).wait()

    def p1_group(g, lvl, res, w_v, idx_v):
        lvl_base = lvl * _T
        p0 = [None] * 3
        for d in range(3):
            x = plsc.load_gather(xin_v, [lane * 3 + (g * 48 + d)])
            pos = x * res
            t = (pos + 65536.0).astype(jnp.int32) - 65536
            w_v[pl.ds(g * 48 + d * 16, 16)] = pos - t.astype(jnp.float32)
            p0[d] = t
        b0 = p0[1] * _PRIME1
        b1 = b0 + _PRIME1
        c0 = p0[2] * _PRIME2
        c1 = c0 + _PRIME2
        a1 = p0[0] + 1
        e = (p0[0] ^ b0, p0[0] ^ b1, a1 ^ b0, a1 ^ b1)
        for o in range(8):
            i, j, k = (o >> 2) & 1, (o >> 1) & 1, o & 1
            h = e[2 * i + j] ^ (c1 if k else c0)
            idx_v[pl.ds(o * _CHUNK + g * 16, 16)] = (h & (_T - 1)) + lvl_base

    def p2_group(g, lvl, w_v, rows_v):
        wx = w_v[pl.ds(g * 48, 16)]
        wy = w_v[pl.ds(g * 48 + 16, 16)]
        wz = w_v[pl.ds(g * 48 + 32, 16)]
        ux = 1.0 - wx
        uy = 1.0 - wy
        uz = 1.0 - wz
        wxy = (ux * uy, ux * wy, wx * uy, wx * wy)
        acc0 = jnp.zeros((16,), jnp.float32)
        acc1 = jnp.zeros((16,), jnp.float32)
        for o in range(8):
            i, j, k = (o >> 2) & 1, (o >> 1) & 1, o & 1
            wt = wxy[2 * i + j] * (wz if k else uz)
            fw = rows_v[pl.ds(o * _CHUNK + g * 16, 16)]
            f0 = plsc.bitcast(fw << 16, jnp.float32)
            f1 = plsc.bitcast(fw & -65536, jnp.float32)
            acc0 = acc0 + wt * f0
            acc1 = acc1 + wt * f1
        eidx = (g * 16 + lane) * _ENC + 2 * lvl
        plsc.store_scatter(enc_v, [eidx], acc0)
        plsc.store_scatter(enc_v, [eidx + 1], acc1)

    def fused(lvl, bcur, bprev2, with_p2):
        res = plsc.load_gather(res_v, [zero16 + lvl])

        def body(g, cc):
            p1_group(g, lvl, res, bufs[bcur][0], bufs[bcur][1])
            if with_p2:
                p2_group(g, lvl - 2, bufs[bprev2][0], bufs[bprev2][2])
            return cc
        lax.fori_loop(0, _NGRP, body, 0)

    def p2_only(lvl, b):
        def body(g, cc):
            p2_group(g, lvl, bufs[b][0], bufs[b][2])
            return cc
        lax.fori_loop(0, _NGRP, body, 0)

    def chunk_body(ci, carry):
        base = wid * _PPW + ci * _CHUNK
        pltpu.sync_copy(xyz_hbm.at[pl.ds(base * 3, _CHUNK * 3)], xin_v)

        def xin_body(i, c):
            v = xin_v[pl.ds(i * 16, 16)]
            xin_v[pl.ds(i * 16, 16)] = ((v + 1.0) / 2.0) * 2.0 - 1.0
            return c
        lax.fori_loop(0, (_CHUNK * 3) // 16, xin_body, 0)

        fused(0, 0, 0, False)
        fire(0)
        fused(1, 1, 0, False)
        fire(1)
        drain(0)

        def tri_body(ii, c):
            lvl = 3 * ii + 2
            for s in range(3):
                b = (2 + s) % 3
                fused(lvl + s, b, (b + 1) % 3, True)
                fire(b)
                drain((b + 2) % 3)
            return c
        lax.fori_loop(0, (_N_LEVELS - 2) // 3, tri_body, 0)

        drain((_N_LEVELS - 1) % 3)
        p2_only(_N_LEVELS - 2, (_N_LEVELS - 2) % 3)
        p2_only(_N_LEVELS - 1, (_N_LEVELS - 1) % 3)

        pltpu.sync_copy(enc_v, enc_hbm.at[pl.ds(base * _ENC, _CHUNK * _ENC)])
        return carry
    lax.fori_loop(0, _NCHUNK, chunk_body, 0)


_encode = functools.partial(
    pl.kernel,
    out_type=jax.ShapeDtypeStruct((_N * _ENC,), jnp.float32),
    mesh=plsc.VectorSubcoreMesh(core_axis_name="c", subcore_axis_name="s"),
    compiler_params=pltpu.CompilerParams(
        needs_layout_passes=False, use_tc_tiling_on_sc=False),
    scratch_types=[
        pltpu.VMEM((_CHUNK * 3,), jnp.float32),
        pltpu.VMEM((_CHUNK * 3,), jnp.float32),
        pltpu.VMEM((_CHUNK * 3,), jnp.float32),
        pltpu.VMEM((_CHUNK * 3,), jnp.float32),
        pltpu.VMEM((_NIDX,), jnp.int32),
        pltpu.VMEM((_NIDX,), jnp.int32),
        pltpu.VMEM((_NIDX,), jnp.int32),
        pltpu.VMEM((_NIDX,), jnp.int32),
        pltpu.VMEM((_NIDX,), jnp.int32),
        pltpu.VMEM((_NIDX,), jnp.int32),
        pltpu.VMEM((_CHUNK * _ENC,), jnp.float32),
        pltpu.VMEM((128,), jnp.float32),
        pltpu.SemaphoreType.DMA,
        pltpu.SemaphoreType.DMA,
        pltpu.SemaphoreType.DMA,
    ],
)(_enc_body)


_BLK = 2048


def _mlp_body(enc_ref, w0_ref, w1_ref, w2_ref, out_ref):
    h = jnp.maximum(
        jnp.dot(enc_ref[...], w0_ref[...], preferred_element_type=jnp.float32), 0.0)
    h = jnp.maximum(
        jnp.dot(h, w1_ref[...], preferred_element_type=jnp.float32), 0.0)
    y = jnp.dot(h, w2_ref[...], preferred_element_type=jnp.float32)
    s = jax.nn.sigmoid(y)
    col = lax.broadcasted_iota(jnp.int32, s.shape, 1)
    out_ref[...] = jnp.where(col == 3, 0.1 + 0.9 * s, s)


def _mlp(enc, W0, W1, W2p):
    return pl.pallas_call(
        _mlp_body,
        grid=(_N // _BLK,),
        in_specs=[
            pl.BlockSpec((_BLK, _ENC), lambda i: (i, 0)),
            pl.BlockSpec((_ENC, 64), lambda i: (0, 0)),
            pl.BlockSpec((64, 64), lambda i: (0, 0)),
            pl.BlockSpec((64, 8), lambda i: (0, 0)),
        ],
        out_specs=pl.BlockSpec((_BLK, 8), lambda i: (i, 0)),
        out_shape=jax.ShapeDtypeStruct((_N, 8), jnp.float32),
    )(enc, W0, W1, W2p)


def kernel(xyz, table, W0, W1, W2):
    xyz_flat = xyz.reshape(-1)
    tt = table.transpose(0, 2, 1)
    a = lax.bitcast_convert_type(
        tt[:, 0, :].astype(jnp.bfloat16), jnp.uint16).astype(jnp.uint32)
    b = lax.bitcast_convert_type(
        tt[:, 1, :].astype(jnp.bfloat16), jnp.uint16).astype(jnp.uint32)
    tabp = lax.bitcast_convert_type(a | (b << 16), jnp.int32).reshape(-1)
    res_arr = jnp.zeros((128,), jnp.float32).at[:_N_LEVELS].set(
        jnp.asarray(_RES, jnp.float32))
    enc = _encode(xyz_flat, tabp, res_arr).reshape(_N, _ENC)
    W2p = jnp.pad(W2, ((0, 0), (0, 3)))
    out = _mlp(enc, W0, W1, W2p)
    return (out[:, 0:3], out[:, 3:4], out[:, 4:5])


# speedup vs baseline: 10.5602x; 1.1658x over previous
import functools

import jax
import jax.numpy as jnp
import numpy as np
from jax import lax
from jax.experimental import pallas as pl
from jax.experimental.pallas import tpu as pltpu
from jax.experimental.pallas import tpu_sc as plsc

_N_LEVELS = 32
_F = 2
_T = 1 << 19
_ENC = 64
_N = 262144

_PRIME1 = -1640531535
_PRIME2 = 805459861
_RES = [float(np.floor(16.0 * 1.3 ** l)) for l in range(_N_LEVELS)]

_NC = 2
_NS = 16
_NW = _NC * _NS
_PPW = _N // _NW
_CHUNK = 256
_NCHUNK = _PPW // _CHUNK
_NGRP = _CHUNK // 16
_NIDX = 8 * _CHUNK
_IDX_ROWS = _NIDX // 128


def _enc_body(xyz_hbm, tab_hbm, res_hbm, enc_hbm,
              xin_v, wA_v, wB_v, wC_v, idxA_v, idxB_v, idxC_v,
              rowsA_v, rowsB_v, rowsC_v, enc_v, res_v, semA, semB, semC):
    cid = lax.axis_index("c")
    sid = lax.axis_index("s")
    wid = sid * _NC + cid
    lane = lax.iota(jnp.int32, 16)
    zero16 = jnp.zeros((16,), jnp.int32)

    pltpu.sync_copy(res_hbm, res_v)

    bufs = ((wA_v, idxA_v, rowsA_v, semA),
            (wB_v, idxB_v, rowsB_v, semB),
            (wC_v, idxC_v, rowsC_v, semC))

    def fire(b):
        pltpu.make_async_copy(
            tab_hbm.at[bufs[b][1]], bufs[b][2], bufs[b][3]).start()

    def drain(b):
        pltpu.make_async_copy(
            tab_hbm.at[bufs[b][1]], bufs[b][2], bufs[b][3]).wait()

    def p1_group(g, lvl, res, w_v, idx_v):
        lvl_base = lvl * _T
        p0 = [None] * 3
        for d in range(3):
            x = xin_v[pl.ds(d * _CHUNK + g * 16, 16)]
            pos = x * res
            t = (pos + 65536.0).astype(jnp.int32) - 65536
            w_v[pl.ds(d * _CHUNK + g * 16, 16)] = pos - t.astype(jnp.float32)
            p0[d] = t
        b0 = p0[1] * _PRIME1
        b1 = b0 + _PRIME1
        c0 = p0[2] * _PRIME2
        c1 = c0 + _PRIME2
        a1 = p0[0] + 1
        e = (p0[0] ^ b0, p0[0] ^ b1, a1 ^ b0, a1 ^ b1)
        for o in range(8):
            i, j, k = (o >> 2) & 1, (o >> 1) & 1, o & 1
            h = e[2 * i + j] ^ (c1 if k else c0)
            idx_v[pl.ds(o * _CHUNK + g * 16, 16)] = (h & (_T - 1)) + lvl_base

    def p2_group(g, lvl, w_v, rows_v):
        wx = w_v[pl.ds(g * 16, 16)]
        wy = w_v[pl.ds(_CHUNK + g * 16, 16)]
        wz = w_v[pl.ds(2 * _CHUNK + g * 16, 16)]
        ux = 1.0 - wx
        uy = 1.0 - wy
        uz = 1.0 - wz
        wxy = (ux * uy, ux * wy, wx * uy, wx * wy)
        acc0 = jnp.zeros((16,), jnp.float32)
        acc1 = jnp.zeros((16,), jnp.float32)
        for o in range(8):
            i, j, k = (o >> 2) & 1, (o >> 1) & 1, o & 1
            wt = wxy[2 * i + j] * (wz if k else uz)
            fw = rows_v[pl.ds(o * _CHUNK + g * 16, 16)]
            f0 = plsc.bitcast(fw << 16, jnp.float32)
            f1 = plsc.bitcast(fw & -65536, jnp.float32)
            acc0 = acc0 + wt * f0
            acc1 = acc1 + wt * f1
        eidx = (g * 16 + lane) * _ENC + 2 * lvl
        plsc.store_scatter(enc_v, [eidx], acc0)
        plsc.store_scatter(enc_v, [eidx + 1], acc1)

    def fused(lvl, bcur, bprev2, with_p2):
        res = plsc.load_gather(res_v, [zero16 + lvl])

        def body(g, cc):
            p1_group(g, lvl, res, bufs[bcur][0], bufs[bcur][1])
            if with_p2:
                p2_group(g, lvl - 2, bufs[bprev2][0], bufs[bprev2][2])
            return cc
        lax.fori_loop(0, _NGRP, body, 0)

    def p2_only(lvl, b):
        def body(g, cc):
            p2_group(g, lvl, bufs[b][0], bufs[b][2])
            return cc
        lax.fori_loop(0, _NGRP, body, 0)

    def chunk_body(ci, carry):
        base = wid * _PPW + ci * _CHUNK
        for d in range(3):
            pltpu.sync_copy(xyz_hbm.at[pl.ds(d * _N + base, _CHUNK)],
                            xin_v.at[pl.ds(d * _CHUNK, _CHUNK)])

        def xin_body(i, c):
            v = xin_v[pl.ds(i * 16, 16)]
            xin_v[pl.ds(i * 16, 16)] = ((v + 1.0) / 2.0) * 2.0 - 1.0
            return c
        lax.fori_loop(0, (_CHUNK * 3) // 16, xin_body, 0)

        fused(0, 0, 0, False)
        fire(0)
        fused(1, 1, 0, False)
        fire(1)
        drain(0)

        def tri_body(ii, c):
            lvl = 3 * ii + 2
            for s in range(3):
                b = (2 + s) % 3
                fused(lvl + s, b, (b + 1) % 3, True)
                fire(b)
                drain((b + 2) % 3)
            return c
        lax.fori_loop(0, (_N_LEVELS - 2) // 3, tri_body, 0)

        drain((_N_LEVELS - 1) % 3)
        p2_only(_N_LEVELS - 2, (_N_LEVELS - 2) % 3)
        p2_only(_N_LEVELS - 1, (_N_LEVELS - 1) % 3)

        pltpu.sync_copy(enc_v, enc_hbm.at[pl.ds(base * _ENC, _CHUNK * _ENC)])
        return carry
    lax.fori_loop(0, _NCHUNK, chunk_body, 0)


_encode = functools.partial(
    pl.kernel,
    out_type=jax.ShapeDtypeStruct((_N * _ENC,), jnp.float32),
    mesh=plsc.VectorSubcoreMesh(core_axis_name="c", subcore_axis_name="s"),
    compiler_params=pltpu.CompilerParams(
        needs_layout_passes=False, use_tc_tiling_on_sc=False),
    scratch_types=[
        pltpu.VMEM((_CHUNK * 3,), jnp.float32),
        pltpu.VMEM((_CHUNK * 3,), jnp.float32),
        pltpu.VMEM((_CHUNK * 3,), jnp.float32),
        pltpu.VMEM((_CHUNK * 3,), jnp.float32),
        pltpu.VMEM((_NIDX,), jnp.int32),
        pltpu.VMEM((_NIDX,), jnp.int32),
        pltpu.VMEM((_NIDX,), jnp.int32),
        pltpu.VMEM((_NIDX,), jnp.int32),
        pltpu.VMEM((_NIDX,), jnp.int32),
        pltpu.VMEM((_NIDX,), jnp.int32),
        pltpu.VMEM((_CHUNK * _ENC,), jnp.float32),
        pltpu.VMEM((128,), jnp.float32),
        pltpu.SemaphoreType.DMA,
        pltpu.SemaphoreType.DMA,
        pltpu.SemaphoreType.DMA,
    ],
)(_enc_body)


_BLK = 2048


def _mlp_body(enc_ref, w0_ref, w1_ref, w2_ref, out_ref):
    h = jnp.maximum(
        jnp.dot(enc_ref[...], w0_ref[...], preferred_element_type=jnp.float32), 0.0)
    h = jnp.maximum(
        jnp.dot(h, w1_ref[...], preferred_element_type=jnp.float32), 0.0)
    y = jnp.dot(h, w2_ref[...], preferred_element_type=jnp.float32)
    s = jax.nn.sigmoid(y)
    col = lax.broadcasted_iota(jnp.int32, s.shape, 1)
    out_ref[...] = jnp.where(col == 3, 0.1 + 0.9 * s, s).T


def _mlp(enc, W0, W1, W2p):
    return pl.pallas_call(
        _mlp_body,
        grid=(_N // _BLK,),
        in_specs=[
            pl.BlockSpec((_BLK, _ENC), lambda i: (i, 0)),
            pl.BlockSpec((_ENC, 64), lambda i: (0, 0)),
            pl.BlockSpec((64, 64), lambda i: (0, 0)),
            pl.BlockSpec((64, 8), lambda i: (0, 0)),
        ],
        out_specs=pl.BlockSpec((8, _BLK), lambda i: (0, i)),
        out_shape=jax.ShapeDtypeStruct((8, _N), jnp.float32),
    )(enc, W0, W1, W2p)


def kernel(xyz, table, W0, W1, W2):
    xyz_flat = xyz.transpose(1, 0).reshape(-1)
    tt = table.transpose(0, 2, 1)
    a = lax.bitcast_convert_type(
        tt[:, 0, :].astype(jnp.bfloat16), jnp.uint16).astype(jnp.uint32)
    b = lax.bitcast_convert_type(
        tt[:, 1, :].astype(jnp.bfloat16), jnp.uint16).astype(jnp.uint32)
    tabp = lax.bitcast_convert_type(a | (b << 16), jnp.int32).reshape(-1)
    res_arr = jnp.zeros((128,), jnp.float32).at[:_N_LEVELS].set(
        jnp.asarray(_RES, jnp.float32))
    enc = _encode(xyz_flat, tabp, res_arr).reshape(_N, _ENC)
    W2p = jnp.pad(W2, ((0, 0), (0, 3)))
    out_t = _mlp(enc, W0, W1, W2p)
    return (out_t[0:3].transpose(1, 0),
            out_t[3:4].transpose(1, 0),
            out_t[4:5].transpose(1, 0))


# speedup vs baseline: 10.6146x; 1.0052x over previous
import functools

import jax
import jax.numpy as jnp
import numpy as np
from jax import lax
from jax.experimental import pallas as pl
from jax.experimental.pallas import tpu as pltpu
from jax.experimental.pallas import tpu_sc as plsc

_N_LEVELS = 32
_F = 2
_T = 1 << 19
_ENC = 64
_N = 262144

_PRIME1 = -1640531535
_PRIME2 = 805459861
_RES = [float(np.floor(16.0 * 1.3 ** l)) for l in range(_N_LEVELS)]

_NC = 2
_NS = 16
_NW = _NC * _NS
_PPW = _N // _NW
_CHUNK = 256
_NCHUNK = _PPW // _CHUNK
_NGRP = _CHUNK // 16
_NIDX = 8 * _CHUNK
_IDX_ROWS = _NIDX // 128


def _enc_body(xyz_hbm, tab_hbm, res_hbm, enc_hbm,
              xin_v, wA_v, wB_v, wC_v, idxA_v, idxB_v, idxC_v,
              rowsA_v, rowsB_v, rowsC_v, enc_v, res_v, semA, semB, semC):
    cid = lax.axis_index("c")
    sid = lax.axis_index("s")
    wid = sid * _NC + cid
    lane = lax.iota(jnp.int32, 16)
    zero16 = jnp.zeros((16,), jnp.int32)

    pltpu.sync_copy(res_hbm, res_v)

    bufs = ((wA_v, idxA_v, rowsA_v, semA),
            (wB_v, idxB_v, rowsB_v, semB),
            (wC_v, idxC_v, rowsC_v, semC))

    def fire(b):
        pltpu.make_async_copy(
            tab_hbm.at[bufs[b][1]], bufs[b][2], bufs[b][3]).start()

    def drain(b):
        pltpu.make_async_copy(
            tab_hbm.at[bufs[b][1]], bufs[b][2], bufs[b][3]).wait()

    def p1_group(g, lvl, res, w_v, idx_v):
        lvl_base = lvl * _T
        p0 = [None] * 3
        for d in range(3):
            x = xin_v[pl.ds(d * _CHUNK + g * 16, 16)]
            pos = x * res
            t = (pos + 65536.0).astype(jnp.int32) - 65536
            w_v[pl.ds(d * _CHUNK + g * 16, 16)] = pos - t.astype(jnp.float32)
            p0[d] = t
        b0 = p0[1] * _PRIME1
        b1 = b0 + _PRIME1
        c0 = p0[2] * _PRIME2
        c1 = c0 + _PRIME2
        a1 = p0[0] + 1
        e = (p0[0] ^ b0, p0[0] ^ b1, a1 ^ b0, a1 ^ b1)
        for o in range(8):
            i, j, k = (o >> 2) & 1, (o >> 1) & 1, o & 1
            h = e[2 * i + j] ^ (c1 if k else c0)
            idx_v[pl.ds(o * _CHUNK + g * 16, 16)] = (h & (_T - 1)) + lvl_base

    def p2_group(g, lvl, w_v, rows_v):
        wx = w_v[pl.ds(g * 16, 16)]
        wy = w_v[pl.ds(_CHUNK + g * 16, 16)]
        wz = w_v[pl.ds(2 * _CHUNK + g * 16, 16)]
        ux = 1.0 - wx
        uy = 1.0 - wy
        uz = 1.0 - wz
        wxy = (ux * uy, ux * wy, wx * uy, wx * wy)
        acc0 = jnp.zeros((16,), jnp.float32)
        acc1 = jnp.zeros((16,), jnp.float32)
        for o in range(8):
            i, j, k = (o >> 2) & 1, (o >> 1) & 1, o & 1
            wt = wxy[2 * i + j] * (wz if k else uz)
            fw = rows_v[pl.ds(o * _CHUNK + g * 16, 16)]
            f0 = plsc.bitcast(fw << 16, jnp.float32)
            f1 = plsc.bitcast(fw & -65536, jnp.float32)
            acc0 = acc0 + wt * f0
            acc1 = acc1 + wt * f1
        eidx = (g * 16 + lane) * _ENC + 2 * lvl
        plsc.store_scatter(enc_v, [eidx], acc0)
        plsc.store_scatter(enc_v, [eidx + 1], acc1)

    def fused(lvl, bcur, bprev2, with_p2):
        res = plsc.load_gather(res_v, [zero16 + lvl])

        def body(g, cc):
            p1_group(g, lvl, res, bufs[bcur][0], bufs[bcur][1])
            if with_p2:
                p2_group(g, lvl - 2, bufs[bprev2][0], bufs[bprev2][2])
            return cc
        lax.fori_loop(0, _NGRP, body, 0)

    def p2_only(lvl, b):
        def body(g, cc):
            p2_group(g, lvl, bufs[b][0], bufs[b][2])
            return cc
        lax.fori_loop(0, _NGRP, body, 0)

    def chunk_body(ci, carry):
        base = wid * _PPW + ci * _CHUNK
        for d in range(3):
            pltpu.sync_copy(xyz_hbm.at[pl.ds(d * _N + base, _CHUNK)],
                            xin_v.at[pl.ds(d * _CHUNK, _CHUNK)])

        def xin_body(i, c):
            v = xin_v[pl.ds(i * 16, 16)]
            xin_v[pl.ds(i * 16, 16)] = ((v + 1.0) / 2.0) * 2.0 - 1.0
            return c
        lax.fori_loop(0, (_CHUNK * 3) // 16, xin_body, 0)

        fused(0, 0, 0, False)
        fire(0)
        fused(1, 1, 0, False)
        fire(1)
        drain(0)

        def tri_body(ii, c):
            lvl = 3 * ii + 2
            for s in range(3):
                b = (2 + s) % 3
                fused(lvl + s, b, (b + 1) % 3, True)
                fire(b)
                drain((b + 2) % 3)
            return c
        lax.fori_loop(0, (_N_LEVELS - 2) // 3, tri_body, 0)

        drain((_N_LEVELS - 1) % 3)
        p2_only(_N_LEVELS - 2, (_N_LEVELS - 2) % 3)
        p2_only(_N_LEVELS - 1, (_N_LEVELS - 1) % 3)

        pltpu.sync_copy(enc_v, enc_hbm.at[pl.ds(base * _ENC, _CHUNK * _ENC)])
        return carry
    lax.fori_loop(0, _NCHUNK, chunk_body, 0)


_encode = functools.partial(
    pl.kernel,
    out_type=jax.ShapeDtypeStruct((_N * _ENC,), jnp.float32),
    mesh=plsc.VectorSubcoreMesh(core_axis_name="c", subcore_axis_name="s"),
    compiler_params=pltpu.CompilerParams(
        needs_layout_passes=False, use_tc_tiling_on_sc=False),
    scratch_types=[
        pltpu.VMEM((_CHUNK * 3,), jnp.float32),
        pltpu.VMEM((_CHUNK * 3,), jnp.float32),
        pltpu.VMEM((_CHUNK * 3,), jnp.float32),
        pltpu.VMEM((_CHUNK * 3,), jnp.float32),
        pltpu.VMEM((_NIDX,), jnp.int32),
        pltpu.VMEM((_NIDX,), jnp.int32),
        pltpu.VMEM((_NIDX,), jnp.int32),
        pltpu.VMEM((_NIDX,), jnp.int32),
        pltpu.VMEM((_NIDX,), jnp.int32),
        pltpu.VMEM((_NIDX,), jnp.int32),
        pltpu.VMEM((_CHUNK * _ENC,), jnp.float32),
        pltpu.VMEM((128,), jnp.float32),
        pltpu.SemaphoreType.DMA,
        pltpu.SemaphoreType.DMA,
        pltpu.SemaphoreType.DMA,
    ],
)(_enc_body)


_BLK = 2048


def _mlp_body(enc_ref, w0_ref, w1_ref, w2_ref, out_ref):
    h = jnp.maximum(
        jnp.dot(enc_ref[...].astype(jnp.bfloat16), w0_ref[...],
                preferred_element_type=jnp.float32), 0.0)
    h = jnp.maximum(
        jnp.dot(h.astype(jnp.bfloat16), w1_ref[...],
                preferred_element_type=jnp.float32), 0.0)
    y = jnp.dot(h.astype(jnp.bfloat16), w2_ref[...],
                preferred_element_type=jnp.float32)
    s = jax.nn.sigmoid(y)
    col = lax.broadcasted_iota(jnp.int32, s.shape, 1)
    out_ref[...] = jnp.where(col == 3, 0.1 + 0.9 * s, s).T


def _mlp(enc, W0, W1, W2p):
    return pl.pallas_call(
        _mlp_body,
        grid=(_N // _BLK,),
        in_specs=[
            pl.BlockSpec((_BLK, _ENC), lambda i: (i, 0)),
            pl.BlockSpec((_ENC, 64), lambda i: (0, 0)),
            pl.BlockSpec((64, 64), lambda i: (0, 0)),
            pl.BlockSpec((64, 8), lambda i: (0, 0)),
        ],
        out_specs=pl.BlockSpec((8, _BLK), lambda i: (0, i)),
        out_shape=jax.ShapeDtypeStruct((8, _N), jnp.float32),
    )(enc, W0, W1, W2p)


def kernel(xyz, table, W0, W1, W2):
    xyz_flat = xyz.transpose(1, 0).reshape(-1)
    ti = lax.bitcast_convert_type(table, jnp.uint32).transpose(0, 2, 1)

    def _rnd(u):
        return (u + jnp.uint32(0x7FFF) + ((u >> 16) & jnp.uint32(1))) >> 16

    tabp = lax.bitcast_convert_type(
        _rnd(ti[:, 0, :]) | (_rnd(ti[:, 1, :]) << 16), jnp.int32).reshape(-1)
    res_arr = jnp.zeros((128,), jnp.float32).at[:_N_LEVELS].set(
        jnp.asarray(_RES, jnp.float32))
    enc = _encode(xyz_flat, tabp, res_arr).reshape(_N, _ENC)
    W2p = jnp.pad(W2, ((0, 0), (0, 3))).astype(jnp.bfloat16)
    out_t = _mlp(enc, W0.astype(jnp.bfloat16), W1.astype(jnp.bfloat16), W2p)
    return (out_t[0:3].transpose(1, 0),
            out_t[3:4].transpose(1, 0),
            out_t[4:5].transpose(1, 0))


# speedup vs baseline: 10.7348x; 1.0113x over previous
import functools

import jax
import jax.numpy as jnp
import numpy as np
from jax import lax
from jax.experimental import pallas as pl
from jax.experimental.pallas import tpu as pltpu
from jax.experimental.pallas import tpu_sc as plsc

_N_LEVELS = 32
_F = 2
_T = 1 << 19
_ENC = 64
_N = 262144

_PRIME1 = -1640531535
_PRIME2 = 805459861
_RES = [float(np.floor(16.0 * 1.3 ** l)) for l in range(_N_LEVELS)]

_NC = 2
_NS = 16
_NW = _NC * _NS
_PPW = _N // _NW
_CHUNK = 256
_NCHUNK = _PPW // _CHUNK
_NGRP = _CHUNK // 16
_NIDX = 8 * _CHUNK
_IDX_ROWS = _NIDX // 128


def _enc_body(n_pts, xyz_hbm, tab_hbm, res_hbm, enc_hbm,
              xin_v, wA_v, wB_v, wC_v, idxA_v, idxB_v, idxC_v,
              rowsA_v, rowsB_v, rowsC_v, enc_v, res_v, semA, semB, semC):
    ppw = n_pts // _NW
    nchunk = ppw // _CHUNK
    cid = lax.axis_index("c")
    sid = lax.axis_index("s")
    wid = sid * _NC + cid
    lane = lax.iota(jnp.int32, 16)
    zero16 = jnp.zeros((16,), jnp.int32)

    pltpu.sync_copy(res_hbm, res_v)

    bufs = ((wA_v, idxA_v, rowsA_v, semA),
            (wB_v, idxB_v, rowsB_v, semB),
            (wC_v, idxC_v, rowsC_v, semC))

    def fire(b):
        pltpu.make_async_copy(
            tab_hbm.at[bufs[b][1]], bufs[b][2], bufs[b][3]).start()

    def drain(b):
        pltpu.make_async_copy(
            tab_hbm.at[bufs[b][1]], bufs[b][2], bufs[b][3]).wait()

    def p1_group(g, lvl, res, w_v, idx_v):
        lvl_base = lvl * _T
        p0 = [None] * 3
        for d in range(3):
            x = xin_v[pl.ds(d * _CHUNK + g * 16, 16)]
            pos = x * res
            t = (pos + 65536.0).astype(jnp.int32) - 65536
            w_v[pl.ds(d * _CHUNK + g * 16, 16)] = pos - t.astype(jnp.float32)
            p0[d] = t
        b0 = p0[1] * _PRIME1
        b1 = b0 + _PRIME1
        c0 = p0[2] * _PRIME2
        c1 = c0 + _PRIME2
        a1 = p0[0] + 1
        e = (p0[0] ^ b0, p0[0] ^ b1, a1 ^ b0, a1 ^ b1)
        for o in range(8):
            i, j, k = (o >> 2) & 1, (o >> 1) & 1, o & 1
            h = e[2 * i + j] ^ (c1 if k else c0)
            idx_v[pl.ds(o * _CHUNK + g * 16, 16)] = (h & (_T - 1)) + lvl_base

    def p2_group(g, lvl, w_v, rows_v):
        wx = w_v[pl.ds(g * 16, 16)]
        wy = w_v[pl.ds(_CHUNK + g * 16, 16)]
        wz = w_v[pl.ds(2 * _CHUNK + g * 16, 16)]
        ux = 1.0 - wx
        uy = 1.0 - wy
        uz = 1.0 - wz
        wxy = (ux * uy, ux * wy, wx * uy, wx * wy)
        acc0 = jnp.zeros((16,), jnp.float32)
        acc1 = jnp.zeros((16,), jnp.float32)
        for o in range(8):
            i, j, k = (o >> 2) & 1, (o >> 1) & 1, o & 1
            wt = wxy[2 * i + j] * (wz if k else uz)
            fw = rows_v[pl.ds(o * _CHUNK + g * 16, 16)]
            f0 = plsc.bitcast(fw << 16, jnp.float32)
            f1 = plsc.bitcast(fw & -65536, jnp.float32)
            acc0 = acc0 + wt * f0
            acc1 = acc1 + wt * f1
        eidx = (g * 16 + lane) * _ENC + 2 * lvl
        plsc.store_scatter(enc_v, [eidx], acc0)
        plsc.store_scatter(enc_v, [eidx + 1], acc1)

    def fused(lvl, bcur, bprev2, with_p2):
        res = plsc.load_gather(res_v, [zero16 + lvl])

        def body(g, cc):
            p1_group(g, lvl, res, bufs[bcur][0], bufs[bcur][1])
            if with_p2:
                p2_group(g, lvl - 2, bufs[bprev2][0], bufs[bprev2][2])
            return cc
        lax.fori_loop(0, _NGRP, body, 0)

    def p2_only(lvl, b):
        def body(g, cc):
            p2_group(g, lvl, bufs[b][0], bufs[b][2])
            return cc
        lax.fori_loop(0, _NGRP, body, 0)

    def chunk_body(ci, carry):
        base = wid * ppw + ci * _CHUNK
        for d in range(3):
            pltpu.sync_copy(xyz_hbm.at[pl.ds(d * n_pts + base, _CHUNK)],
                            xin_v.at[pl.ds(d * _CHUNK, _CHUNK)])

        def xin_body(i, c):
            v = xin_v[pl.ds(i * 16, 16)]
            xin_v[pl.ds(i * 16, 16)] = ((v + 1.0) / 2.0) * 2.0 - 1.0
            return c
        lax.fori_loop(0, (_CHUNK * 3) // 16, xin_body, 0)

        fused(0, 0, 0, False)
        fire(0)
        fused(1, 1, 0, False)
        fire(1)
        drain(0)

        def tri_body(ii, c):
            lvl = 3 * ii + 2
            for s in range(3):
                b = (2 + s) % 3
                fused(lvl + s, b, (b + 1) % 3, True)
                fire(b)
                drain((b + 2) % 3)
            return c
        lax.fori_loop(0, (_N_LEVELS - 2) // 3, tri_body, 0)

        drain((_N_LEVELS - 1) % 3)
        p2_only(_N_LEVELS - 2, (_N_LEVELS - 2) % 3)
        p2_only(_N_LEVELS - 1, (_N_LEVELS - 1) % 3)

        pltpu.sync_copy(enc_v, enc_hbm.at[pl.ds(base * _ENC, _CHUNK * _ENC)])
        return carry
    lax.fori_loop(0, nchunk, chunk_body, 0)


def _make_encode(n_pts):
    return functools.partial(
        pl.kernel,
        out_type=jax.ShapeDtypeStruct((n_pts * _ENC,), jnp.float32),
        mesh=plsc.VectorSubcoreMesh(core_axis_name="c", subcore_axis_name="s"),
        compiler_params=pltpu.CompilerParams(
            needs_layout_passes=False, use_tc_tiling_on_sc=False),
    scratch_types=[
        pltpu.VMEM((_CHUNK * 3,), jnp.float32),
        pltpu.VMEM((_CHUNK * 3,), jnp.float32),
        pltpu.VMEM((_CHUNK * 3,), jnp.float32),
        pltpu.VMEM((_CHUNK * 3,), jnp.float32),
        pltpu.VMEM((_NIDX,), jnp.int32),
        pltpu.VMEM((_NIDX,), jnp.int32),
        pltpu.VMEM((_NIDX,), jnp.int32),
        pltpu.VMEM((_NIDX,), jnp.int32),
        pltpu.VMEM((_NIDX,), jnp.int32),
        pltpu.VMEM((_NIDX,), jnp.int32),
        pltpu.VMEM((_CHUNK * _ENC,), jnp.float32),
        pltpu.VMEM((128,), jnp.float32),
        pltpu.SemaphoreType.DMA,
        pltpu.SemaphoreType.DMA,
        pltpu.SemaphoreType.DMA,
    ],
    )(functools.partial(_enc_body, n_pts))


_BLK = 2048


def _mlp_body(enc_ref, w0_ref, w1_ref, w2_ref, out_ref):
    h = jnp.maximum(
        jnp.dot(enc_ref[...].astype(jnp.bfloat16), w0_ref[...],
                preferred_element_type=jnp.float32), 0.0)
    h = jnp.maximum(
        jnp.dot(h.astype(jnp.bfloat16), w1_ref[...],
                preferred_element_type=jnp.float32), 0.0)
    y = jnp.dot(h.astype(jnp.bfloat16), w2_ref[...],
                preferred_element_type=jnp.float32)
    s = jax.nn.sigmoid(y)
    col = lax.broadcasted_iota(jnp.int32, s.shape, 1)
    out_ref[...] = jnp.where(col == 3, 0.1 + 0.9 * s, s).T


def _mlp(enc, W0, W1, W2p):
    n = enc.shape[0]
    return pl.pallas_call(
        _mlp_body,
        grid=(n // _BLK,),
        in_specs=[
            pl.BlockSpec((_BLK, _ENC), lambda i: (i, 0)),
            pl.BlockSpec((_ENC, 64), lambda i: (0, 0)),
            pl.BlockSpec((64, 64), lambda i: (0, 0)),
            pl.BlockSpec((64, 8), lambda i: (0, 0)),
        ],
        out_specs=pl.BlockSpec((8, _BLK), lambda i: (0, i)),
        out_shape=jax.ShapeDtypeStruct((8, n), jnp.float32),
    )(enc, W0, W1, W2p)


_HALF = _N // 2
_encode_half = _make_encode(_HALF)


def kernel(xyz, table, W0, W1, W2):
    xyzT = xyz.transpose(1, 0)
    ti = lax.bitcast_convert_type(table, jnp.uint32).transpose(0, 2, 1)

    def _rnd(u):
        return (u + jnp.uint32(0x7FFF) + ((u >> 16) & jnp.uint32(1))) >> 16

    tabp = lax.bitcast_convert_type(
        _rnd(ti[:, 0, :]) | (_rnd(ti[:, 1, :]) << 16), jnp.int32).reshape(-1)
    res_arr = jnp.zeros((128,), jnp.float32).at[:_N_LEVELS].set(
        jnp.asarray(_RES, jnp.float32))
    W2p = jnp.pad(W2, ((0, 0), (0, 3))).astype(jnp.bfloat16)
    W0b = W0.astype(jnp.bfloat16)
    W1b = W1.astype(jnp.bfloat16)
    outs = []
    for h in range(2):
        xh = xyzT[:, h * _HALF:(h + 1) * _HALF].reshape(-1)
        ench = _encode_half(xh, tabp, res_arr).reshape(_HALF, _ENC)
        outs.append(_mlp(ench, W0b, W1b, W2p))
    out_t = jnp.concatenate(outs, axis=1)
    return (out_t[0:3].transpose(1, 0),
            out_t[3:4].transpose(1, 0),
            out_t[4:5].transpose(1, 0))
